# Initial kernel scaffold; baseline (speedup 1.0000x reference)
import jax
import jax.numpy as jnp
from jax.experimental import pallas as pl


def kernel(edge_feat, edge_index, edge_mask, num_ligand_atoms, node_feat, angle, dists, struct_tab, plip_lig, plip_prot, plip_inter, dist_W1, dist_b1, dist_W2, dist_b2, gtvd, ang_W1, ang_b1, ang_W2, ang_b2, md_W1, md_b1, md_W2, md_b2):
    raise NotImplementedError("write your pallas kernel here")


# speedup vs baseline: 2.1921x; 2.1921x over previous
import functools

import jax
import jax.numpy as jnp
from jax import lax
from jax.experimental import pallas as pl
from jax.experimental.pallas import tpu as pltpu
from jax.experimental.pallas import tpu_sc as plsc

B, N, E, H = 4, 384, 8192, 32
NP1 = N + 1
RB = 32
KB = N // RB
HG = 4
HPG = H // HG
CELLS = N * N
ACC_FLAT = HPG * CELLS
NTILES = 16
TILE_E = E // NTILES
ZCH = 9216


def _edge_emb_body(ef_ref, ei_ref, m_ref, nl_ref, stab_ref, plig_ref,
                   pprot_ref, pinter_ref, dw1_ref, db1_ref, dw2_ref,
                   db2_ref, out_ref):
    x = ef_ref[0]
    et0 = x[0:1, :].astype(jnp.int32)
    et1 = x[1:2, :].astype(jnp.int32)
    et2 = x[2:3, :].astype(jnp.int32)
    d = x[3:4, :]
    ei = ei_ref[0]
    src = ei[0:1, :]
    tgt = ei[1:2, :]
    nl = jnp.maximum(nl_ref[0], 1)
    src_l = (src > 0) & (src < nl)
    tgt_l = (tgt > 0) & (tgt < nl)

    sidx = jnp.clip(et0 * 4 + et1 * 2 + et2, 0, 19)
    ohs = (lax.broadcasted_iota(jnp.int32, (H, E), 0) == sidx).astype(jnp.float32)
    semb = jnp.dot(stab_ref[...], ohs, preferred_element_type=jnp.float32)

    pidx = jnp.clip(et1, 0, 14)
    ohp = (lax.broadcasted_iota(jnp.int32, (16, E), 0) == pidx).astype(jnp.float32)
    p_lig = jnp.dot(plig_ref[...], ohp, preferred_element_type=jnp.float32)
    p_prot = jnp.dot(pprot_ref[...], ohp, preferred_element_type=jnp.float32)
    p_int = jnp.dot(pinter_ref[...], ohp, preferred_element_type=jnp.float32)
    both_lig = src_l & tgt_l
    both_prot = (~src_l) & (~tgt_l)
    pemb = jnp.where(both_lig, p_lig, jnp.where(both_prot, p_prot, p_int))

    h1 = jax.nn.relu(dw1_ref[...] * d + db1_ref[...])
    demb = jnp.dot(dw2_ref[...], h1, preferred_element_type=jnp.float32) + db2_ref[...]

    is_struct = et0 <= 1
    is_plip = et0 == 5
    sel = jnp.where(is_struct, semb, jnp.where(is_plip, pemb, jnp.zeros_like(demb)))
    out_ref[0] = (sel + demb) * m_ref[0]


def _edge_emb(ef_t, ei, mask_f, nlig, stabT, pligT, pprotT, pinterT,
              dw1, db1, dw2, db2):
    full = lambda shape: pl.BlockSpec(shape, lambda b: tuple(0 for _ in shape))
    return pl.pallas_call(
        _edge_emb_body,
        grid=(B,),
        in_specs=[
            pl.BlockSpec((1, 4, E), lambda b: (b, 0, 0)),
            pl.BlockSpec((1, 2, E), lambda b: (b, 0, 0)),
            pl.BlockSpec((1, 1, E), lambda b: (b, 0, 0)),
            pl.BlockSpec((1, 1, 1), lambda b: (b, 0, 0)),
            full((H, H)), full((H, 16)), full((H, 16)), full((H, 16)),
            full((H, 1)), full((H, 1)), full((H, H)), full((H, 1)),
        ],
        out_specs=pl.BlockSpec((1, H, E), lambda b: (b, 0, 0)),
        out_shape=jax.ShapeDtypeStruct((B, H, E), jnp.float32),
    )(ef_t, ei, mask_f, nlig, stabT, pligT, pprotT, pinterT, dw1, db1, dw2, db2)


def _sc_scatter_body(embT_hbm, ei_hbm, out_hbm, src_v, tgt_v, emb_v, idx_v,
                     val_v, zbuf, acc_sh):
    c = lax.axis_index("c")
    s = lax.axis_index("s")
    base = s * (ACC_FLAT // NTILES)

    def _zb(i, carry):
        zbuf[pl.ds(i * 16, 16)] = jnp.zeros((16,), jnp.float32)
        return carry
    lax.fori_loop(0, ZCH // 16, _zb, 0)

    for i in range(B * HG // 2):
        p = 2 * i + c
        b = p // HG
        hg = p % HG

        def _zero(j, carry):
            pltpu.sync_copy(zbuf, acc_sh.at[pl.ds(base + j * ZCH, ZCH)])
            return carry
        lax.fori_loop(0, (ACC_FLAT // NTILES) // ZCH, _zero, 0)
        plsc.subcore_barrier()

        e0 = s * TILE_E
        pltpu.sync_copy(ei_hbm.at[b, 0, pl.ds(e0, TILE_E)], src_v)
        pltpu.sync_copy(ei_hbm.at[b, 1, pl.ds(e0, TILE_E)], tgt_v)
        pltpu.sync_copy(
            embT_hbm.at[b, pl.ds(hg * HPG, HPG), pl.ds(e0, TILE_E)], emb_v)

        def _chunk(ci, carry):
            sv = src_v[pl.ds(ci * 16, 16)]
            tv = tgt_v[pl.ds(ci * 16, 16)]
            cell = sv * N + tv
            for h in range(HPG):
                idx_v[pl.ds(h * 16, 16)] = cell + h * CELLS
                val_v[pl.ds(h * 16, 16)] = emb_v[h, pl.ds(ci * 16, 16)]
            pltpu.sync_copy(val_v, acc_sh.at[idx_v], add=True)
            return carry
        lax.fori_loop(0, TILE_E // 16, _chunk, 0)
        plsc.subcore_barrier()

        def _out(j, carry):
            off = base + j * ZCH
            pltpu.sync_copy(acc_sh.at[pl.ds(off, ZCH)],
                            out_hbm.at[b, hg, pl.ds(off, ZCH)])
            return carry
        lax.fori_loop(0, (ACC_FLAT // NTILES) // ZCH, _out, 0)
        plsc.subcore_barrier()


@functools.lru_cache(maxsize=1)
def _sc_scatter_kernel():
    mesh = plsc.VectorSubcoreMesh(core_axis_name="c", subcore_axis_name="s")
    return pl.kernel(
        _sc_scatter_body,
        mesh=mesh,
        out_type=jax.ShapeDtypeStruct((B, HG, ACC_FLAT), jnp.float32),
        scratch_types=[
            pltpu.VMEM((TILE_E,), jnp.int32),
            pltpu.VMEM((TILE_E,), jnp.int32),
            pltpu.VMEM((HPG, TILE_E), jnp.float32),
            pltpu.VMEM((128,), jnp.int32),
            pltpu.VMEM((128,), jnp.float32),
            pltpu.VMEM((ZCH,), jnp.float32),
            pltpu.VMEM_SHARED((ACC_FLAT,), jnp.float32),
        ],
    )


def _dense_body(ang_ref, dst_ref, acc_ref, aw1, ab1, aw2, ab2, mw1, mb1,
                mw2, mb2, gt_ref, out_ref, obuf, fbuf, carry_ref, sem, sem0):
    b = pl.program_id(0)
    k = pl.program_id(1)
    Xa = ang_ref[0].reshape(RB * N, 28)
    Xd = dst_ref[0].reshape(RB * N, 28)
    h1 = jax.nn.relu(jnp.dot(Xa, aw1[...], preferred_element_type=jnp.float32)
                     + ab1[...])
    pa = jnp.dot(h1, aw2[...], preferred_element_type=jnp.float32) + ab2[...]
    h2 = jax.nn.relu(jnp.dot(Xd, mw1[...], preferred_element_type=jnp.float32)
                     + mb1[...])
    pd = jnp.dot(h2, mw2[...], preferred_element_type=jnp.float32) + mb2[...]
    pair = pa + pd
    pT3 = pair.T.reshape(H, RB, N)
    tot = pT3 + acc_ref[0]
    t = gt_ref[...]

    @pl.when(k == 0)
    def _():
        carry_ref[...] = jnp.broadcast_to(t[:, :, None], (H, 1, N))

    interior = jnp.concatenate([carry_ref[...], tot[:, :RB - 1, :]], axis=1)
    col0 = jnp.broadcast_to(t[:, :, None], (H, RB, 1))
    obuf[...] = jnp.concatenate([col0, interior], axis=2)
    carry_ref[...] = tot[:, RB - 1:RB, :]
    cp = pltpu.make_async_copy(
        obuf, out_ref.at[b, :, pl.ds(k * RB, RB), :], sem)
    cp.start()
    cp.wait()

    @pl.when(k == KB - 1)
    def _():
        fbuf[...] = jnp.concatenate(
            [t[:, :, None], tot[:, RB - 1:RB, :]], axis=2)
        cp0 = pltpu.make_async_copy(fbuf, out_ref.at[b, :, pl.ds(N, 1), :],
                                    sem0)
        cp0.start()
        cp0.wait()


def _dense(angle, dists, acc4, aw1, ab1, aw2, ab2, mw1, mb1, mw2, mb2, gtc):
    wfull = lambda shape: pl.BlockSpec(shape, lambda b, k: tuple(0 for _ in shape))
    return pl.pallas_call(
        _dense_body,
        grid=(B, KB),
        in_specs=[
            pl.BlockSpec((1, RB, N, 28), lambda b, k: (b, k, 0, 0)),
            pl.BlockSpec((1, RB, N, 28), lambda b, k: (b, k, 0, 0)),
            pl.BlockSpec((1, H, RB, N), lambda b, k: (b, 0, k, 0)),
            wfull((28, H)), wfull((1, H)), wfull((H, H)), wfull((1, H)),
            wfull((28, H)), wfull((1, H)), wfull((H, H)), wfull((1, H)),
            wfull((H, 1)),
        ],
        out_specs=pl.BlockSpec(memory_space=pl.ANY),
        out_shape=jax.ShapeDtypeStruct((B, H, NP1, NP1), jnp.float32),
        scratch_shapes=[
            pltpu.VMEM((H, RB, NP1), jnp.float32),
            pltpu.VMEM((H, 1, NP1), jnp.float32),
            pltpu.VMEM((H, 1, N), jnp.float32),
            pltpu.SemaphoreType.DMA,
            pltpu.SemaphoreType.DMA,
        ],
    )(angle, dists, acc4, aw1, ab1, aw2, ab2, mw1, mb1, mw2, mb2, gtc)


def kernel(edge_feat, edge_index, edge_mask, num_ligand_atoms, node_feat,
           angle, dists, struct_tab, plip_lig, plip_prot, plip_inter,
           dist_W1, dist_b1, dist_W2, dist_b2, gtvd, ang_W1, ang_b1,
           ang_W2, ang_b2, md_W1, md_b1, md_W2, md_b2):
    f32 = jnp.float32
    ef_t = jnp.transpose(edge_feat, (0, 2, 1)).astype(f32)
    ei = edge_index.astype(jnp.int32)
    mask_f = edge_mask[:, None, :].astype(f32)
    nlig = num_ligand_atoms.astype(jnp.int32)[:, None, None]
    stabT = jnp.concatenate([struct_tab, jnp.zeros((12, H), f32)], axis=0).T
    pligT = jnp.concatenate([plip_lig, jnp.zeros((1, H), f32)], axis=0).T
    pprotT = jnp.concatenate([plip_prot, jnp.zeros((1, H), f32)], axis=0).T
    pinterT = jnp.concatenate([plip_inter, jnp.zeros((1, H), f32)], axis=0).T

    embT = _edge_emb(ef_t, ei, mask_f, nlig, stabT, pligT, pprotT, pinterT,
                     dist_W1, dist_b1.reshape(H, 1), dist_W2,
                     dist_b2.reshape(H, 1))

    acc = _sc_scatter_kernel()(embT, ei)
    acc4 = acc.reshape(B, H, N, N)

    gb = _dense(angle, dists, acc4, ang_W1.T, ang_b1.reshape(1, H),
                ang_W2.T, ang_b2.reshape(1, H), md_W1.T,
                md_b1.reshape(1, H), md_W2.T, md_b2.reshape(1, H),
                gtvd.reshape(H, 1))
    return gb


# speedup vs baseline: 2.2365x; 1.0203x over previous
import functools

import jax
import jax.numpy as jnp
from jax import lax
from jax.experimental import pallas as pl
from jax.experimental.pallas import tpu as pltpu
from jax.experimental.pallas import tpu_sc as plsc

B, N, E, H = 4, 384, 8192, 32
NP1 = N + 1
RB = 32
KB = N // RB
HG = 4
HPG = H // HG
CELLS = N * N
ACC_FLAT = HPG * CELLS
NTILES = 16
TILE_E = E // NTILES
ZCH = 9216


def _edge_emb_body(ef_ref, ei_ref, m_ref, nl_ref, stab_ref, plig_ref,
                   pprot_ref, pinter_ref, dw1_ref, db1_ref, dw2_ref,
                   db2_ref, out_ref):
    x = ef_ref[0]
    et0 = x[0:1, :].astype(jnp.int32)
    et1 = x[1:2, :].astype(jnp.int32)
    et2 = x[2:3, :].astype(jnp.int32)
    d = x[3:4, :]
    ei = ei_ref[0]
    src = ei[0:1, :]
    tgt = ei[1:2, :]
    nl = jnp.maximum(nl_ref[0], 1)
    src_l = (src > 0) & (src < nl)
    tgt_l = (tgt > 0) & (tgt < nl)

    sidx = jnp.clip(et0 * 4 + et1 * 2 + et2, 0, 19)
    ohs = (lax.broadcasted_iota(jnp.int32, (H, E), 0) == sidx).astype(jnp.float32)
    semb = jnp.dot(stab_ref[...], ohs, preferred_element_type=jnp.float32)

    pidx = jnp.clip(et1, 0, 14)
    ohp = (lax.broadcasted_iota(jnp.int32, (16, E), 0) == pidx).astype(jnp.float32)
    p_lig = jnp.dot(plig_ref[...], ohp, preferred_element_type=jnp.float32)
    p_prot = jnp.dot(pprot_ref[...], ohp, preferred_element_type=jnp.float32)
    p_int = jnp.dot(pinter_ref[...], ohp, preferred_element_type=jnp.float32)
    both_lig = src_l & tgt_l
    both_prot = (~src_l) & (~tgt_l)
    pemb = jnp.where(both_lig, p_lig, jnp.where(both_prot, p_prot, p_int))

    h1 = jax.nn.relu(dw1_ref[...] * d + db1_ref[...])
    demb = jnp.dot(dw2_ref[...], h1, preferred_element_type=jnp.float32) + db2_ref[...]

    is_struct = et0 <= 1
    is_plip = et0 == 5
    sel = jnp.where(is_struct, semb, jnp.where(is_plip, pemb, jnp.zeros_like(demb)))
    out_ref[0] = (sel + demb) * m_ref[0]


def _edge_emb(ef_t, ei, mask_f, nlig, stabT, pligT, pprotT, pinterT,
              dw1, db1, dw2, db2):
    full = lambda shape: pl.BlockSpec(shape, lambda b: tuple(0 for _ in shape))
    return pl.pallas_call(
        _edge_emb_body,
        grid=(B,),
        in_specs=[
            pl.BlockSpec((1, 4, E), lambda b: (b, 0, 0)),
            pl.BlockSpec((1, 2, E), lambda b: (b, 0, 0)),
            pl.BlockSpec((1, 1, E), lambda b: (b, 0, 0)),
            pl.BlockSpec((1, 1, 1), lambda b: (b, 0, 0)),
            full((H, H)), full((H, 16)), full((H, 16)), full((H, 16)),
            full((H, 1)), full((H, 1)), full((H, H)), full((H, 1)),
        ],
        out_specs=pl.BlockSpec((1, H, E), lambda b: (b, 0, 0)),
        out_shape=jax.ShapeDtypeStruct((B, H, E), jnp.float32),
    )(ef_t, ei, mask_f, nlig, stabT, pligT, pprotT, pinterT, dw1, db1, dw2, db2)


def _sc_scatter_body(embT_hbm, ei_hbm, out_hbm, src_v, tgt_v, emb_v, idx_v,
                     val_v, zbuf, acc_sh):
    c = lax.axis_index("c")
    s = lax.axis_index("s")
    base = s * (ACC_FLAT // NTILES)

    def _zb(i, carry):
        zbuf[pl.ds(i * 16, 16)] = jnp.zeros((16,), jnp.float32)
        return carry
    lax.fori_loop(0, ZCH // 16, _zb, 0)

    for i in range(B * HG // 2):
        p = 2 * i + c
        b = p // HG
        hg = p % HG

        def _zero(j, carry):
            pltpu.sync_copy(zbuf, acc_sh.at[pl.ds(base + j * ZCH, ZCH)])
            return carry
        lax.fori_loop(0, (ACC_FLAT // NTILES) // ZCH, _zero, 0)
        plsc.subcore_barrier()

        e0 = s * TILE_E
        pltpu.sync_copy(ei_hbm.at[b, 0, pl.ds(e0, TILE_E)], src_v)
        pltpu.sync_copy(ei_hbm.at[b, 1, pl.ds(e0, TILE_E)], tgt_v)
        pltpu.sync_copy(
            embT_hbm.at[b, pl.ds(hg * HPG, HPG), pl.ds(e0, TILE_E)], emb_v)

        def _chunk(ci, carry):
            sv = src_v[pl.ds(ci * 16, 16)]
            tv = tgt_v[pl.ds(ci * 16, 16)]
            cell = sv * N + tv
            for h in range(HPG):
                idx_v[pl.ds(h * 16, 16)] = cell + h * CELLS
                val_v[pl.ds(h * 16, 16)] = emb_v[h, pl.ds(ci * 16, 16)]
            pltpu.sync_copy(val_v, acc_sh.at[idx_v], add=True)
            return carry
        lax.fori_loop(0, TILE_E // 16, _chunk, 0)
        plsc.subcore_barrier()

        def _out(j, carry):
            off = base + j * ZCH
            pltpu.sync_copy(acc_sh.at[pl.ds(off, ZCH)],
                            out_hbm.at[b, hg, pl.ds(off, ZCH)])
            return carry
        lax.fori_loop(0, (ACC_FLAT // NTILES) // ZCH, _out, 0)
        plsc.subcore_barrier()


@functools.lru_cache(maxsize=1)
def _sc_scatter_kernel():
    mesh = plsc.VectorSubcoreMesh(core_axis_name="c", subcore_axis_name="s")
    return pl.kernel(
        _sc_scatter_body,
        mesh=mesh,
        out_type=jax.ShapeDtypeStruct((B, HG, ACC_FLAT), jnp.float32),
        scratch_types=[
            pltpu.VMEM((TILE_E,), jnp.int32),
            pltpu.VMEM((TILE_E,), jnp.int32),
            pltpu.VMEM((HPG, TILE_E), jnp.float32),
            pltpu.VMEM((128,), jnp.int32),
            pltpu.VMEM((128,), jnp.float32),
            pltpu.VMEM((ZCH,), jnp.float32),
            pltpu.VMEM_SHARED((ACC_FLAT,), jnp.float32),
        ],
    )


def _dense_body(ang_ref, dst_ref, acc_ref, aw1, ab1, aw2, ab2, mw1, mb1,
                mw2, mb2, gt_ref, out_ref, obuf, fbuf, carry_ref, sem, sem0):
    b = pl.program_id(0)
    k = pl.program_id(1)
    Xa = ang_ref[0].reshape(RB * N, 28)
    Xd = dst_ref[0].reshape(RB * N, 28)
    h1 = jax.nn.relu(jnp.dot(Xa, aw1[...], preferred_element_type=jnp.float32)
                     + ab1[...])
    pa = jnp.dot(h1, aw2[...], preferred_element_type=jnp.float32) + ab2[...]
    h2 = jax.nn.relu(jnp.dot(Xd, mw1[...], preferred_element_type=jnp.float32)
                     + mb1[...])
    pd = jnp.dot(h2, mw2[...], preferred_element_type=jnp.float32) + mb2[...]
    pair = pa + pd
    pT3 = pair.T.reshape(H, RB, N)
    tot = pT3 + acc_ref[0]
    t = gt_ref[...]

    @pl.when(k == 0)
    def _():
        carry_ref[...] = jnp.broadcast_to(t[:, :, None], (H, 1, N))

    interior = jnp.concatenate([carry_ref[...], tot[:, :RB - 1, :]], axis=1)
    col0 = jnp.broadcast_to(t[:, :, None], (H, RB, 1))
    n = b * KB + k
    slot = lax.rem(n, 2)

    @pl.when(n >= 2)
    def _():
        pltpu.make_async_copy(obuf.at[slot], out_ref.at[0, :, pl.ds(0, RB), :],
                              sem).wait()

    obuf[slot] = jnp.concatenate([col0, interior], axis=2)
    carry_ref[...] = tot[:, RB - 1:RB, :]
    pltpu.make_async_copy(
        obuf.at[slot], out_ref.at[b, :, pl.ds(k * RB, RB), :], sem).start()

    @pl.when(k == KB - 1)
    def _():
        fbuf[...] = jnp.concatenate(
            [t[:, :, None], tot[:, RB - 1:RB, :]], axis=2)
        cp0 = pltpu.make_async_copy(fbuf, out_ref.at[b, :, pl.ds(N, 1), :],
                                    sem0)
        cp0.start()
        cp0.wait()

    @pl.when(n == B * KB - 1)
    def _():
        pltpu.make_async_copy(obuf.at[slot], out_ref.at[0, :, pl.ds(0, RB), :],
                              sem).wait()
        pltpu.make_async_copy(obuf.at[slot], out_ref.at[0, :, pl.ds(0, RB), :],
                              sem).wait()


def _dense(angle, dists, acc4, aw1, ab1, aw2, ab2, mw1, mb1, mw2, mb2, gtc):
    wfull = lambda shape: pl.BlockSpec(shape, lambda b, k: tuple(0 for _ in shape))
    return pl.pallas_call(
        _dense_body,
        grid=(B, KB),
        in_specs=[
            pl.BlockSpec((1, RB, N, 28), lambda b, k: (b, k, 0, 0)),
            pl.BlockSpec((1, RB, N, 28), lambda b, k: (b, k, 0, 0)),
            pl.BlockSpec((1, H, RB, N), lambda b, k: (b, 0, k, 0)),
            wfull((28, H)), wfull((1, H)), wfull((H, H)), wfull((1, H)),
            wfull((28, H)), wfull((1, H)), wfull((H, H)), wfull((1, H)),
            wfull((H, 1)),
        ],
        out_specs=pl.BlockSpec(memory_space=pl.ANY),
        out_shape=jax.ShapeDtypeStruct((B, H, NP1, NP1), jnp.float32),
        scratch_shapes=[
            pltpu.VMEM((2, H, RB, NP1), jnp.float32),
            pltpu.VMEM((H, 1, NP1), jnp.float32),
            pltpu.VMEM((H, 1, N), jnp.float32),
            pltpu.SemaphoreType.DMA,
            pltpu.SemaphoreType.DMA,
        ],
    )(angle, dists, acc4, aw1, ab1, aw2, ab2, mw1, mb1, mw2, mb2, gtc)


def kernel(edge_feat, edge_index, edge_mask, num_ligand_atoms, node_feat,
           angle, dists, struct_tab, plip_lig, plip_prot, plip_inter,
           dist_W1, dist_b1, dist_W2, dist_b2, gtvd, ang_W1, ang_b1,
           ang_W2, ang_b2, md_W1, md_b1, md_W2, md_b2):
    f32 = jnp.float32
    ef_t = jnp.transpose(edge_feat, (0, 2, 1)).astype(f32)
    ei = edge_index.astype(jnp.int32)
    mask_f = edge_mask[:, None, :].astype(f32)
    nlig = num_ligand_atoms.astype(jnp.int32)[:, None, None]
    stabT = jnp.concatenate([struct_tab, jnp.zeros((12, H), f32)], axis=0).T
    pligT = jnp.concatenate([plip_lig, jnp.zeros((1, H), f32)], axis=0).T
    pprotT = jnp.concatenate([plip_prot, jnp.zeros((1, H), f32)], axis=0).T
    pinterT = jnp.concatenate([plip_inter, jnp.zeros((1, H), f32)], axis=0).T

    embT = _edge_emb(ef_t, ei, mask_f, nlig, stabT, pligT, pprotT, pinterT,
                     dist_W1, dist_b1.reshape(H, 1), dist_W2,
                     dist_b2.reshape(H, 1))

    acc = _sc_scatter_kernel()(embT, ei)
    acc4 = acc.reshape(B, H, N, N)

    gb = _dense(angle, dists, acc4, ang_W1.T, ang_b1.reshape(1, H),
                ang_W2.T, ang_b2.reshape(1, H), md_W1.T,
                md_b1.reshape(1, H), md_W2.T, md_b2.reshape(1, H),
                gtvd.reshape(H, 1))
    return gb


# speedup vs baseline: 2.2510x; 1.0065x over previous
import functools

import jax
import jax.numpy as jnp
from jax import lax
from jax.experimental import pallas as pl
from jax.experimental.pallas import tpu as pltpu
from jax.experimental.pallas import tpu_sc as plsc

B, N, E, H = 4, 384, 8192, 32
NP1 = N + 1
RB = 48
KB = N // RB
HG = 4
HPG = H // HG
CELLS = N * N
ACC_FLAT = HPG * CELLS
NTILES = 16
TILE_E = E // NTILES
ZCH = 9216


def _edge_emb_body(ef_ref, ei_ref, m_ref, nl_ref, stab_ref, plig_ref,
                   pprot_ref, pinter_ref, dw1_ref, db1_ref, dw2_ref,
                   db2_ref, out_ref):
    x = ef_ref[0]
    et0 = x[0:1, :].astype(jnp.int32)
    et1 = x[1:2, :].astype(jnp.int32)
    et2 = x[2:3, :].astype(jnp.int32)
    d = x[3:4, :]
    ei = ei_ref[0]
    src = ei[0:1, :]
    tgt = ei[1:2, :]
    nl = jnp.maximum(nl_ref[0], 1)
    src_l = (src > 0) & (src < nl)
    tgt_l = (tgt > 0) & (tgt < nl)

    sidx = jnp.clip(et0 * 4 + et1 * 2 + et2, 0, 19)
    ohs = (lax.broadcasted_iota(jnp.int32, (H, E), 0) == sidx).astype(jnp.float32)
    semb = jnp.dot(stab_ref[...], ohs, preferred_element_type=jnp.float32)

    pidx = jnp.clip(et1, 0, 14)
    ohp = (lax.broadcasted_iota(jnp.int32, (16, E), 0) == pidx).astype(jnp.float32)
    p_lig = jnp.dot(plig_ref[...], ohp, preferred_element_type=jnp.float32)
    p_prot = jnp.dot(pprot_ref[...], ohp, preferred_element_type=jnp.float32)
    p_int = jnp.dot(pinter_ref[...], ohp, preferred_element_type=jnp.float32)
    both_lig = src_l & tgt_l
    both_prot = (~src_l) & (~tgt_l)
    pemb = jnp.where(both_lig, p_lig, jnp.where(both_prot, p_prot, p_int))

    h1 = jax.nn.relu(dw1_ref[...] * d + db1_ref[...])
    demb = jnp.dot(dw2_ref[...], h1, preferred_element_type=jnp.float32) + db2_ref[...]

    is_struct = et0 <= 1
    is_plip = et0 == 5
    sel = jnp.where(is_struct, semb, jnp.where(is_plip, pemb, jnp.zeros_like(demb)))
    out_ref[0] = (sel + demb) * m_ref[0]


def _edge_emb(ef_t, ei, mask_f, nlig, stabT, pligT, pprotT, pinterT,
              dw1, db1, dw2, db2):
    full = lambda shape: pl.BlockSpec(shape, lambda b: tuple(0 for _ in shape))
    return pl.pallas_call(
        _edge_emb_body,
        grid=(B,),
        in_specs=[
            pl.BlockSpec((1, 4, E), lambda b: (b, 0, 0)),
            pl.BlockSpec((1, 2, E), lambda b: (b, 0, 0)),
            pl.BlockSpec((1, 1, E), lambda b: (b, 0, 0)),
            pl.BlockSpec((1, 1, 1), lambda b: (b, 0, 0)),
            full((H, H)), full((H, 16)), full((H, 16)), full((H, 16)),
            full((H, 1)), full((H, 1)), full((H, H)), full((H, 1)),
        ],
        out_specs=pl.BlockSpec((1, H, E), lambda b: (b, 0, 0)),
        out_shape=jax.ShapeDtypeStruct((B, H, E), jnp.float32),
    )(ef_t, ei, mask_f, nlig, stabT, pligT, pprotT, pinterT, dw1, db1, dw2, db2)


def _sc_scatter_body(embT_hbm, ei_hbm, out_hbm, src_v, tgt_v, emb_v, idx_v,
                     val_v, zbuf, acc_sh):
    c = lax.axis_index("c")
    s = lax.axis_index("s")
    base = s * (ACC_FLAT // NTILES)

    def _zb(i, carry):
        zbuf[pl.ds(i * 16, 16)] = jnp.zeros((16,), jnp.float32)
        return carry
    lax.fori_loop(0, ZCH // 16, _zb, 0)

    for i in range(B * HG // 2):
        p = 2 * i + c
        b = p // HG
        hg = p % HG

        def _zero(j, carry):
            pltpu.sync_copy(zbuf, acc_sh.at[pl.ds(base + j * ZCH, ZCH)])
            return carry
        lax.fori_loop(0, (ACC_FLAT // NTILES) // ZCH, _zero, 0)
        plsc.subcore_barrier()

        e0 = s * TILE_E
        pltpu.sync_copy(ei_hbm.at[b, 0, pl.ds(e0, TILE_E)], src_v)
        pltpu.sync_copy(ei_hbm.at[b, 1, pl.ds(e0, TILE_E)], tgt_v)
        pltpu.sync_copy(
            embT_hbm.at[b, pl.ds(hg * HPG, HPG), pl.ds(e0, TILE_E)], emb_v)

        def _chunk(ci, carry):
            sv = src_v[pl.ds(ci * 16, 16)]
            tv = tgt_v[pl.ds(ci * 16, 16)]
            cell = sv * N + tv
            for h in range(HPG):
                idx_v[pl.ds(h * 16, 16)] = cell + h * CELLS
                val_v[pl.ds(h * 16, 16)] = emb_v[h, pl.ds(ci * 16, 16)]
            pltpu.sync_copy(val_v, acc_sh.at[idx_v], add=True)
            return carry
        lax.fori_loop(0, TILE_E // 16, _chunk, 0)
        plsc.subcore_barrier()

        def _out(j, carry):
            off = base + j * ZCH
            pltpu.sync_copy(acc_sh.at[pl.ds(off, ZCH)],
                            out_hbm.at[b, hg, pl.ds(off, ZCH)])
            return carry
        lax.fori_loop(0, (ACC_FLAT // NTILES) // ZCH, _out, 0)
        plsc.subcore_barrier()


@functools.lru_cache(maxsize=1)
def _sc_scatter_kernel():
    mesh = plsc.VectorSubcoreMesh(core_axis_name="c", subcore_axis_name="s")
    return pl.kernel(
        _sc_scatter_body,
        mesh=mesh,
        out_type=jax.ShapeDtypeStruct((B, HG, ACC_FLAT), jnp.float32),
        scratch_types=[
            pltpu.VMEM((TILE_E,), jnp.int32),
            pltpu.VMEM((TILE_E,), jnp.int32),
            pltpu.VMEM((HPG, TILE_E), jnp.float32),
            pltpu.VMEM((128,), jnp.int32),
            pltpu.VMEM((128,), jnp.float32),
            pltpu.VMEM((ZCH,), jnp.float32),
            pltpu.VMEM_SHARED((ACC_FLAT,), jnp.float32),
        ],
    )


def _dense_body(ang_ref, dst_ref, acc_ref, aw1, ab1, aw2, ab2, mw1, mb1,
                mw2, mb2, gt_ref, out_ref, obuf, fbuf, carry_ref, sem, sem0):
    b = pl.program_id(0)
    k = pl.program_id(1)
    Xa = ang_ref[0].reshape(RB * N, 28)
    Xd = dst_ref[0].reshape(RB * N, 28)
    h1 = jax.nn.relu(jnp.dot(Xa, aw1[...], preferred_element_type=jnp.float32)
                     + ab1[...])
    pa = jnp.dot(h1, aw2[...], preferred_element_type=jnp.float32) + ab2[...]
    h2 = jax.nn.relu(jnp.dot(Xd, mw1[...], preferred_element_type=jnp.float32)
                     + mb1[...])
    pd = jnp.dot(h2, mw2[...], preferred_element_type=jnp.float32) + mb2[...]
    pair = pa + pd
    pT3 = pair.T.reshape(H, RB, N)
    tot = pT3 + acc_ref[0]
    t = gt_ref[...]

    @pl.when(k == 0)
    def _():
        carry_ref[...] = jnp.broadcast_to(t[:, :, None], (H, 1, N))

    interior = jnp.concatenate([carry_ref[...], tot[:, :RB - 1, :]], axis=1)
    col0 = jnp.broadcast_to(t[:, :, None], (H, RB, 1))
    n = b * KB + k
    slot = lax.rem(n, 2)

    @pl.when(n >= 2)
    def _():
        pltpu.make_async_copy(obuf.at[slot], out_ref.at[0, :, pl.ds(0, RB), :],
                              sem).wait()

    obuf[slot] = jnp.concatenate([col0, interior], axis=2)
    carry_ref[...] = tot[:, RB - 1:RB, :]
    pltpu.make_async_copy(
        obuf.at[slot], out_ref.at[b, :, pl.ds(k * RB, RB), :], sem).start()

    @pl.when(k == KB - 1)
    def _():
        fbuf[...] = jnp.concatenate(
            [t[:, :, None], tot[:, RB - 1:RB, :]], axis=2)
        cp0 = pltpu.make_async_copy(fbuf, out_ref.at[b, :, pl.ds(N, 1), :],
                                    sem0)
        cp0.start()
        cp0.wait()

    @pl.when(n == B * KB - 1)
    def _():
        pltpu.make_async_copy(obuf.at[slot], out_ref.at[0, :, pl.ds(0, RB), :],
                              sem).wait()
        pltpu.make_async_copy(obuf.at[slot], out_ref.at[0, :, pl.ds(0, RB), :],
                              sem).wait()


def _dense(angle, dists, acc4, aw1, ab1, aw2, ab2, mw1, mb1, mw2, mb2, gtc):
    wfull = lambda shape: pl.BlockSpec(shape, lambda b, k: tuple(0 for _ in shape))
    return pl.pallas_call(
        _dense_body,
        grid=(B, KB),
        in_specs=[
            pl.BlockSpec((1, RB, N, 28), lambda b, k: (b, k, 0, 0)),
            pl.BlockSpec((1, RB, N, 28), lambda b, k: (b, k, 0, 0)),
            pl.BlockSpec((1, H, RB, N), lambda b, k: (b, 0, k, 0)),
            wfull((28, H)), wfull((1, H)), wfull((H, H)), wfull((1, H)),
            wfull((28, H)), wfull((1, H)), wfull((H, H)), wfull((1, H)),
            wfull((H, 1)),
        ],
        out_specs=pl.BlockSpec(memory_space=pl.ANY),
        out_shape=jax.ShapeDtypeStruct((B, H, NP1, NP1), jnp.float32),
        scratch_shapes=[
            pltpu.VMEM((2, H, RB, NP1), jnp.float32),
            pltpu.VMEM((H, 1, NP1), jnp.float32),
            pltpu.VMEM((H, 1, N), jnp.float32),
            pltpu.SemaphoreType.DMA,
            pltpu.SemaphoreType.DMA,
        ],
    )(angle, dists, acc4, aw1, ab1, aw2, ab2, mw1, mb1, mw2, mb2, gtc)


def kernel(edge_feat, edge_index, edge_mask, num_ligand_atoms, node_feat,
           angle, dists, struct_tab, plip_lig, plip_prot, plip_inter,
           dist_W1, dist_b1, dist_W2, dist_b2, gtvd, ang_W1, ang_b1,
           ang_W2, ang_b2, md_W1, md_b1, md_W2, md_b2):
    f32 = jnp.float32
    ef_t = jnp.transpose(edge_feat, (0, 2, 1)).astype(f32)
    ei = edge_index.astype(jnp.int32)
    mask_f = edge_mask[:, None, :].astype(f32)
    nlig = num_ligand_atoms.astype(jnp.int32)[:, None, None]
    stabT = jnp.concatenate([struct_tab, jnp.zeros((12, H), f32)], axis=0).T
    pligT = jnp.concatenate([plip_lig, jnp.zeros((1, H), f32)], axis=0).T
    pprotT = jnp.concatenate([plip_prot, jnp.zeros((1, H), f32)], axis=0).T
    pinterT = jnp.concatenate([plip_inter, jnp.zeros((1, H), f32)], axis=0).T

    embT = _edge_emb(ef_t, ei, mask_f, nlig, stabT, pligT, pprotT, pinterT,
                     dist_W1, dist_b1.reshape(H, 1), dist_W2,
                     dist_b2.reshape(H, 1))

    acc = _sc_scatter_kernel()(embT, ei)
    acc4 = acc.reshape(B, H, N, N)

    gb = _dense(angle, dists, acc4, ang_W1.T, ang_b1.reshape(1, H),
                ang_W2.T, ang_b2.reshape(1, H), md_W1.T,
                md_b1.reshape(1, H), md_W2.T, md_b2.reshape(1, H),
                gtvd.reshape(H, 1))
    return gb
